# baseline (speedup 1.0000x reference)
import jax
import jax.numpy as jnp
from jax import lax
from jax.experimental import pallas as pl
from jax.experimental.pallas import tpu as pltpu
from jax.experimental.pallas import tpu_sc as plsc

N = 10000
IN_CH = 128
D = 64
HD = D // 2
E = 320000
NC = 2
NS = 16
NW = NC * NS
CH = 128
NCHUNK = E // CH
DEG_Q, DEG_R = divmod(NCHUNK, NW)
SC_Q, SC_R = divmod(NCHUNK, NS)
N_PAD = 10240
RPT = N_PAD // NS
DEPTH = 8
WIN = 16

_mesh = plsc.VectorSubcoreMesh(
    core_axis_name="c", subcore_axis_name="s", num_cores=NC, num_subcores=NS
)
_sc_params = pltpu.CompilerParams(use_tc_tiling_on_sc=False)


def _deg_body(e_hbm, ones_hbm, zeros_hbm, deg_out, idx_v, ones_v, deg_sh, dsem):
    c = lax.axis_index("c")
    s = lax.axis_index("s")
    wid = c * NS + s
    cnt = DEG_Q + jnp.where(wid < DEG_R, 1, 0)
    lo = NCHUNK + wid * DEG_Q + jnp.minimum(wid, DEG_R)
    pltpu.sync_copy(e_hbm.at[pl.ds(lo, DEG_Q), :], idx_v.at[pl.ds(0, DEG_Q), :])

    @pl.when(wid < DEG_R)
    def _():
        pltpu.sync_copy(e_hbm.at[pl.ds(lo + DEG_Q, 1), :],
                        idx_v.at[pl.ds(DEG_Q, 1), :])

    pltpu.sync_copy(ones_hbm, ones_v)
    for q in range(RPT // CH):
        pltpu.sync_copy(zeros_hbm, deg_sh.at[pl.ds(s * RPT + q * CH, CH)])
    plsc.subcore_barrier()

    def body(i, carry):
        for b in range(DEPTH):
            pltpu.async_copy(ones_v, deg_sh.at[idx_v.at[i * DEPTH + b]], dsem,
                             add=True)
        for b in range(DEPTH):
            pltpu.make_async_copy(
                ones_v, deg_sh.at[idx_v.at[i * DEPTH + b]], dsem
            ).wait()
        return carry

    lax.fori_loop(0, DEG_Q // DEPTH, body, 0)

    def tail(j, carry):
        pltpu.sync_copy(ones_v, deg_sh.at[idx_v.at[j]], add=True)
        return carry

    lax.fori_loop((DEG_Q // DEPTH) * DEPTH, cnt, tail, 0)
    plsc.subcore_barrier()
    pltpu.sync_copy(
        deg_sh.at[pl.ds(s * RPT, RPT)],
        deg_out.at[pl.ds(c * N_PAD + s * RPT, RPT)],
    )


_deg_kernel = pl.kernel(
    _deg_body,
    out_type=jax.ShapeDtypeStruct((NC * N_PAD,), jnp.float32),
    mesh=_mesh,
    scratch_types=[
        pltpu.VMEM((DEG_Q + 1, CH), jnp.int32),
        pltpu.VMEM((CH,), jnp.float32),
        pltpu.VMEM_SHARED((N_PAD,), jnp.float32),
        pltpu.SemaphoreType.DMA,
    ],
    compiler_params=_sc_params,
)

_ITER = -(-(SC_Q + 1) // DEPTH)


def _scat_body(e_hbm, g0_hbm, g1_hbm, zeros_hbm, acc_out,
               sidx_v, didx_v, gbuf, g_sh, acc_sh, gsem, ssem):
    c = lax.axis_index("c")
    s = lax.axis_index("s")
    cnt = SC_Q + jnp.where(s < SC_R, 1, 0)
    lo = s * SC_Q + jnp.minimum(s, SC_R)
    pltpu.sync_copy(e_hbm.at[pl.ds(lo, SC_Q), :], sidx_v.at[pl.ds(0, SC_Q), :])
    pltpu.sync_copy(e_hbm.at[pl.ds(NCHUNK + lo, SC_Q), :],
                    didx_v.at[pl.ds(0, SC_Q), :])

    @pl.when(s < SC_R)
    def _():
        pltpu.sync_copy(e_hbm.at[pl.ds(lo + SC_Q, 1), :],
                        sidx_v.at[pl.ds(SC_Q, 1), :])
        pltpu.sync_copy(e_hbm.at[pl.ds(NCHUNK + lo + SC_Q, 1), :],
                        didx_v.at[pl.ds(SC_Q, 1), :])

    for q in range(RPT // CH):
        pltpu.sync_copy(zeros_hbm, acc_sh.at[pl.ds(s * RPT + q * CH, CH), :])
    base = s * RPT

    @pl.when(c == 0)
    def _():
        pltpu.sync_copy(g0_hbm.at[pl.ds(base, RPT), :], g_sh.at[pl.ds(base, RPT), :])

    @pl.when(c == 1)
    def _():
        pltpu.sync_copy(g1_hbm.at[pl.ds(base, RPT), :], g_sh.at[pl.ds(base, RPT), :])

    plsc.subcore_barrier()
    for b in range(DEPTH):
        pltpu.async_copy(g_sh.at[sidx_v.at[b]], gbuf.at[b], gsem.at[b])

    def body(i, carry):
        j0 = i * DEPTH
        for b in range(DEPTH):
            j = j0 + b

            @pl.when(j < cnt)
            def _():
                pltpu.make_async_copy(
                    g_sh.at[sidx_v.at[j]], gbuf.at[b], gsem.at[b]
                ).wait()
                pltpu.async_copy(gbuf.at[b], acc_sh.at[didx_v.at[j]],
                                 ssem.at[b], add=True)

        for b in range(DEPTH):
            j = j0 + b + DEPTH

            @pl.when(j < cnt)
            def _():
                pltpu.make_async_copy(
                    gbuf.at[b], acc_sh.at[didx_v.at[j0 + b]], ssem.at[b]
                ).wait()
                pltpu.async_copy(g_sh.at[sidx_v.at[j]], gbuf.at[b], gsem.at[b])

        return carry

    lax.fori_loop(0, _ITER, body, 0)
    for b in range(DEPTH):
        pltpu.make_async_copy(
            gbuf.at[b], acc_sh.at[didx_v.at[0]], ssem.at[b]
        ).wait()
    plsc.subcore_barrier()
    pltpu.sync_copy(
        acc_sh.at[pl.ds(s * RPT, RPT), :],
        acc_out.at[pl.ds(s * RPT, RPT), pl.ds(c * HD, HD)],
    )


_scat_kernel = pl.kernel(
    _scat_body,
    out_type=jax.ShapeDtypeStruct((N_PAD, D), jnp.float32),
    mesh=_mesh,
    scratch_types=[
        pltpu.VMEM((SC_Q + 1, CH), jnp.int32),
        pltpu.VMEM((SC_Q + 1, CH), jnp.int32),
        pltpu.VMEM((DEPTH, CH, HD), jnp.float32),
        pltpu.VMEM_SHARED((N_PAD, HD), jnp.float32),
        pltpu.VMEM_SHARED((N_PAD, HD), jnp.float32),
        pltpu.SemaphoreType.DMA((DEPTH,)),
        pltpu.SemaphoreType.DMA((DEPTH,)),
    ],
    compiler_params=_sc_params,
)


def _proj_body(x_ref, w_ref, b_ref, deg_ref, g0_ref, g1_ref):
    h = lax.dot_general(
        x_ref[...], w_ref[...], (((1,), (1,)), ((), ())),
        preferred_element_type=jnp.float32,
    )
    h = h + b_ref[...]
    nrm = jnp.sqrt(jnp.sum(h * h, axis=1, keepdims=True))
    h = (h / jnp.maximum(nrm, 1e-12)) * 1.8
    d = deg_ref[pl.ds(0, N), :] + deg_ref[pl.ds(N_PAD, N), :] + 1.0
    g = h * lax.rsqrt(d)
    g0_ref[pl.ds(0, N), :] = g[:, :HD]
    g1_ref[pl.ds(0, N), :] = g[:, HD:]
    tail = jnp.zeros((N_PAD - N, HD), jnp.float32)
    g0_ref[pl.ds(N, N_PAD - N), :] = tail
    g1_ref[pl.ds(N, N_PAD - N), :] = tail


_proj = pl.pallas_call(
    _proj_body,
    out_shape=(
        jax.ShapeDtypeStruct((N_PAD, HD), jnp.float32),
        jax.ShapeDtypeStruct((N_PAD, HD), jnp.float32),
    ),
)


def _out_body(g0_ref, g1_ref, acc_ref, deg_ref, o_ref):
    d = deg_ref[pl.ds(0, N), :] + deg_ref[pl.ds(N_PAD, N), :] + 1.0
    g = jnp.concatenate([g0_ref[pl.ds(0, N), :], g1_ref[pl.ds(0, N), :]], axis=1)
    o_ref[...] = lax.rsqrt(d) * (g + acc_ref[pl.ds(0, N), :])


_out = pl.pallas_call(
    _out_body,
    out_shape=jax.ShapeDtypeStruct((N, D), jnp.float32),
)


def kernel(x, edge_index, W, b):
    e2d = edge_index.astype(jnp.int32).reshape(NC * NCHUNK, CH)
    ones = jnp.ones((CH,), jnp.float32)
    zeros1 = jnp.zeros((CH,), jnp.float32)
    zeros2 = jnp.zeros((CH, HD), jnp.float32)

    degp = _deg_kernel(e2d, ones, zeros1)
    degp2 = degp.reshape(NC * N_PAD, 1)
    g0, g1 = _proj(x, W, b.reshape(1, D), degp2)
    acc = _scat_kernel(e2d, g0, g1, zeros2)
    return _out(g0, g1, acc, degp2)


# speedup vs baseline: 1.0673x; 1.0673x over previous
import jax
import jax.numpy as jnp
from jax import lax
from jax.experimental import pallas as pl
from jax.experimental.pallas import tpu as pltpu
from jax.experimental.pallas import tpu_sc as plsc

N = 10000
IN_CH = 128
D = 64
HD = D // 2
E = 320000
NC = 2
NS = 16
NW = NC * NS
CH = 128
NCHUNK = E // CH
DEG_Q, DEG_R = divmod(NCHUNK, NW)
SC_Q, SC_R = divmod(NCHUNK, NS)
N_PAD = 10240
RPT = N_PAD // NS
DEPTH = 8
WIN = 16

_mesh = plsc.VectorSubcoreMesh(
    core_axis_name="c", subcore_axis_name="s", num_cores=NC, num_subcores=NS
)
_sc_params = pltpu.CompilerParams(use_tc_tiling_on_sc=False)


def _deg_body(e_hbm, ones_hbm, zeros_hbm, deg_out, idx_v, ones_v, deg_sh, dsem):
    c = lax.axis_index("c")
    s = lax.axis_index("s")
    wid = c * NS + s
    cnt = DEG_Q + jnp.where(wid < DEG_R, 1, 0)
    lo = NCHUNK + wid * DEG_Q + jnp.minimum(wid, DEG_R)
    pltpu.sync_copy(e_hbm.at[pl.ds(lo, DEG_Q), :], idx_v.at[pl.ds(0, DEG_Q), :])

    @pl.when(wid < DEG_R)
    def _():
        pltpu.sync_copy(e_hbm.at[pl.ds(lo + DEG_Q, 1), :],
                        idx_v.at[pl.ds(DEG_Q, 1), :])

    pltpu.sync_copy(ones_hbm, ones_v)
    for q in range(RPT // CH):
        pltpu.sync_copy(zeros_hbm, deg_sh.at[pl.ds(s * RPT + q * CH, CH)])
    plsc.subcore_barrier()

    def body(i, carry):
        for b in range(DEPTH):
            pltpu.async_copy(ones_v, deg_sh.at[idx_v.at[i * DEPTH + b]], dsem,
                             add=True)
        for b in range(DEPTH):
            pltpu.make_async_copy(
                ones_v, deg_sh.at[idx_v.at[i * DEPTH + b]], dsem
            ).wait()
        return carry

    lax.fori_loop(0, DEG_Q // DEPTH, body, 0)

    def tail(j, carry):
        pltpu.sync_copy(ones_v, deg_sh.at[idx_v.at[j]], add=True)
        return carry

    lax.fori_loop((DEG_Q // DEPTH) * DEPTH, cnt, tail, 0)
    plsc.subcore_barrier()
    pltpu.sync_copy(
        deg_sh.at[pl.ds(s * RPT, RPT)],
        deg_out.at[pl.ds(c * N_PAD + s * RPT, RPT)],
    )


_deg_kernel = pl.kernel(
    _deg_body,
    out_type=jax.ShapeDtypeStruct((NC * N_PAD,), jnp.float32),
    mesh=_mesh,
    scratch_types=[
        pltpu.VMEM((DEG_Q + 1, CH), jnp.int32),
        pltpu.VMEM((CH,), jnp.float32),
        pltpu.VMEM_SHARED((N_PAD,), jnp.float32),
        pltpu.SemaphoreType.DMA,
    ],
    compiler_params=_sc_params,
)

_ITER = -(-(SC_Q + 1) // DEPTH)


def _scat_body(e_hbm, g0_hbm, g1_hbm, zeros_hbm, acc_out,
               sidx_v, didx_v, gbuf, g_sh, acc_sh, gsem):
    c = lax.axis_index("c")
    s = lax.axis_index("s")
    cnt = SC_Q + jnp.where(s < SC_R, 1, 0)
    lo = s * SC_Q + jnp.minimum(s, SC_R)
    pltpu.sync_copy(e_hbm.at[pl.ds(lo, SC_Q), :], sidx_v.at[pl.ds(0, SC_Q), :])
    pltpu.sync_copy(e_hbm.at[pl.ds(NCHUNK + lo, SC_Q), :],
                    didx_v.at[pl.ds(0, SC_Q), :])

    @pl.when(s < SC_R)
    def _():
        pltpu.sync_copy(e_hbm.at[pl.ds(lo + SC_Q, 1), :],
                        sidx_v.at[pl.ds(SC_Q, 1), :])
        pltpu.sync_copy(e_hbm.at[pl.ds(NCHUNK + lo + SC_Q, 1), :],
                        didx_v.at[pl.ds(SC_Q, 1), :])

    for q in range(RPT // CH):
        pltpu.sync_copy(zeros_hbm, acc_sh.at[pl.ds(s * RPT + q * CH, CH), :])
    base = s * RPT

    @pl.when(c == 0)
    def _():
        pltpu.sync_copy(g0_hbm.at[pl.ds(base, RPT), :], g_sh.at[pl.ds(base, RPT), :])

    @pl.when(c == 1)
    def _():
        pltpu.sync_copy(g1_hbm.at[pl.ds(base, RPT), :], g_sh.at[pl.ds(base, RPT), :])

    plsc.subcore_barrier()
    for b in range(DEPTH):
        pltpu.async_copy(g_sh.at[sidx_v.at[b]], gbuf.at[b], gsem.at[b])

    def body(i, carry):
        j0 = i * DEPTH
        for b in range(DEPTH):
            j = j0 + b

            @pl.when(j < cnt)
            def _():
                pltpu.make_async_copy(
                    g_sh.at[sidx_v.at[j]], gbuf.at[b], gsem.at[b]
                ).wait()
                pltpu.sync_copy(gbuf.at[b], acc_sh.at[didx_v.at[j]], add=True)

            jn = j + DEPTH

            @pl.when(jn < cnt)
            def _():
                pltpu.async_copy(g_sh.at[sidx_v.at[jn]], gbuf.at[b], gsem.at[b])

        return carry

    lax.fori_loop(0, _ITER, body, 0)
    plsc.subcore_barrier()
    pltpu.sync_copy(
        acc_sh.at[pl.ds(s * RPT, RPT), :],
        acc_out.at[pl.ds(s * RPT, RPT), pl.ds(c * HD, HD)],
    )


_scat_kernel = pl.kernel(
    _scat_body,
    out_type=jax.ShapeDtypeStruct((N_PAD, D), jnp.float32),
    mesh=_mesh,
    scratch_types=[
        pltpu.VMEM((SC_Q + 1, CH), jnp.int32),
        pltpu.VMEM((SC_Q + 1, CH), jnp.int32),
        pltpu.VMEM((DEPTH, CH, HD), jnp.float32),
        pltpu.VMEM_SHARED((N_PAD, HD), jnp.float32),
        pltpu.VMEM_SHARED((N_PAD, HD), jnp.float32),
        pltpu.SemaphoreType.DMA((DEPTH,)),
    ],
    compiler_params=_sc_params,
)


def _proj_body(x_ref, w_ref, b_ref, deg_ref, g0_ref, g1_ref):
    h = lax.dot_general(
        x_ref[...], w_ref[...], (((1,), (1,)), ((), ())),
        preferred_element_type=jnp.float32,
    )
    h = h + b_ref[...]
    nrm = jnp.sqrt(jnp.sum(h * h, axis=1, keepdims=True))
    h = (h / jnp.maximum(nrm, 1e-12)) * 1.8
    d = deg_ref[pl.ds(0, N), :] + deg_ref[pl.ds(N_PAD, N), :] + 1.0
    g = h * lax.rsqrt(d)
    g0_ref[pl.ds(0, N), :] = g[:, :HD]
    g1_ref[pl.ds(0, N), :] = g[:, HD:]
    tail = jnp.zeros((N_PAD - N, HD), jnp.float32)
    g0_ref[pl.ds(N, N_PAD - N), :] = tail
    g1_ref[pl.ds(N, N_PAD - N), :] = tail


_proj = pl.pallas_call(
    _proj_body,
    out_shape=(
        jax.ShapeDtypeStruct((N_PAD, HD), jnp.float32),
        jax.ShapeDtypeStruct((N_PAD, HD), jnp.float32),
    ),
)


def _out_body(g0_ref, g1_ref, acc_ref, deg_ref, o_ref):
    d = deg_ref[pl.ds(0, N), :] + deg_ref[pl.ds(N_PAD, N), :] + 1.0
    g = jnp.concatenate([g0_ref[pl.ds(0, N), :], g1_ref[pl.ds(0, N), :]], axis=1)
    o_ref[...] = lax.rsqrt(d) * (g + acc_ref[pl.ds(0, N), :])


_out = pl.pallas_call(
    _out_body,
    out_shape=jax.ShapeDtypeStruct((N, D), jnp.float32),
)


def kernel(x, edge_index, W, b):
    e2d = edge_index.astype(jnp.int32).reshape(NC * NCHUNK, CH)
    ones = jnp.ones((CH,), jnp.float32)
    zeros1 = jnp.zeros((CH,), jnp.float32)
    zeros2 = jnp.zeros((CH, HD), jnp.float32)

    degp = _deg_kernel(e2d, ones, zeros1)
    degp2 = degp.reshape(NC * N_PAD, 1)
    g0, g1 = _proj(x, W, b.reshape(1, D), degp2)
    acc = _scat_kernel(e2d, g0, g1, zeros2)
    return _out(g0, g1, acc, degp2)


# speedup vs baseline: 1.1183x; 1.0478x over previous
import jax
import jax.numpy as jnp
from jax import lax
from jax.experimental import pallas as pl
from jax.experimental.pallas import tpu as pltpu
from jax.experimental.pallas import tpu_sc as plsc

N = 10000
IN_CH = 128
D = 64
HD = D // 2
E = 320000
NC = 2
NS = 16
NW = NC * NS
CH = 128
NCHUNK = E // CH
DEG_Q, DEG_R = divmod(NCHUNK, NW)
SC_Q, SC_R = divmod(NCHUNK, NS)
N_PAD = 10240
RPT = N_PAD // NS
HRO = RPT // 4
TAIL_V = tuple(max(0, min(HRO, N - ((NS - 1) * RPT + q * HRO))) for q in range(4))
DEPTH = 8
WIN = 16

_mesh = plsc.VectorSubcoreMesh(
    core_axis_name="c", subcore_axis_name="s", num_cores=NC, num_subcores=NS
)
_sc_params = pltpu.CompilerParams(use_tc_tiling_on_sc=False)
_sc_params_nl = pltpu.CompilerParams(
    use_tc_tiling_on_sc=False, needs_layout_passes=False
)


def _deg_body(e_hbm, ones_hbm, zeros_hbm, deg_out, idx_v, ones_v, deg_sh, dsem):
    c = lax.axis_index("c")
    s = lax.axis_index("s")
    wid = c * NS + s
    cnt = DEG_Q + jnp.where(wid < DEG_R, 1, 0)
    lo = NCHUNK + wid * DEG_Q + jnp.minimum(wid, DEG_R)
    pltpu.sync_copy(e_hbm.at[pl.ds(lo, DEG_Q), :], idx_v.at[pl.ds(0, DEG_Q), :])

    @pl.when(wid < DEG_R)
    def _():
        pltpu.sync_copy(e_hbm.at[pl.ds(lo + DEG_Q, 1), :],
                        idx_v.at[pl.ds(DEG_Q, 1), :])

    pltpu.sync_copy(ones_hbm, ones_v)
    for q in range(RPT // CH):
        pltpu.sync_copy(zeros_hbm, deg_sh.at[pl.ds(s * RPT + q * CH, CH)])
    plsc.subcore_barrier()

    def body(i, carry):
        for b in range(DEPTH):
            pltpu.async_copy(ones_v, deg_sh.at[idx_v.at[i * DEPTH + b]], dsem,
                             add=True)
        for b in range(DEPTH):
            pltpu.make_async_copy(
                ones_v, deg_sh.at[idx_v.at[i * DEPTH + b]], dsem
            ).wait()
        return carry

    lax.fori_loop(0, DEG_Q // DEPTH, body, 0)

    def tail(j, carry):
        pltpu.sync_copy(ones_v, deg_sh.at[idx_v.at[j]], add=True)
        return carry

    lax.fori_loop((DEG_Q // DEPTH) * DEPTH, cnt, tail, 0)
    plsc.subcore_barrier()
    pltpu.sync_copy(
        deg_sh.at[pl.ds(s * RPT, RPT)],
        deg_out.at[pl.ds(c * N_PAD + s * RPT, RPT)],
    )


_deg_kernel = pl.kernel(
    _deg_body,
    out_type=jax.ShapeDtypeStruct((NC * N_PAD,), jnp.float32),
    mesh=_mesh,
    scratch_types=[
        pltpu.VMEM((DEG_Q + 1, CH), jnp.int32),
        pltpu.VMEM((CH,), jnp.float32),
        pltpu.VMEM_SHARED((N_PAD,), jnp.float32),
        pltpu.SemaphoreType.DMA,
    ],
    compiler_params=_sc_params,
)

_ITER = -(-(SC_Q + 1) // DEPTH)


def _scat_body(e_hbm, g0_hbm, g1_hbm, zeros_hbm, degp_hbm, out_hbm,
               sidx_v, didx_v, gbuf, g_sh, acc_sh, gsem, d0_v, d1_v, gv, av):
    c = lax.axis_index("c")
    s = lax.axis_index("s")
    cnt = SC_Q + jnp.where(s < SC_R, 1, 0)
    lo = s * SC_Q + jnp.minimum(s, SC_R)
    pltpu.sync_copy(e_hbm.at[pl.ds(lo, SC_Q), :], sidx_v.at[pl.ds(0, SC_Q), :])
    pltpu.sync_copy(e_hbm.at[pl.ds(NCHUNK + lo, SC_Q), :],
                    didx_v.at[pl.ds(0, SC_Q), :])

    @pl.when(s < SC_R)
    def _():
        pltpu.sync_copy(e_hbm.at[pl.ds(lo + SC_Q, 1), :],
                        sidx_v.at[pl.ds(SC_Q, 1), :])
        pltpu.sync_copy(e_hbm.at[pl.ds(NCHUNK + lo + SC_Q, 1), :],
                        didx_v.at[pl.ds(SC_Q, 1), :])

    for q in range(RPT // CH):
        pltpu.sync_copy(zeros_hbm, acc_sh.at[pl.ds(s * RPT + q * CH, CH), :])
    base = s * RPT

    @pl.when(c == 0)
    def _():
        pltpu.sync_copy(g0_hbm.at[pl.ds(base, RPT), :], g_sh.at[pl.ds(base, RPT), :])

    @pl.when(c == 1)
    def _():
        pltpu.sync_copy(g1_hbm.at[pl.ds(base, RPT), :], g_sh.at[pl.ds(base, RPT), :])

    plsc.subcore_barrier()
    for b in range(DEPTH):
        pltpu.async_copy(g_sh.at[sidx_v.at[b]], gbuf.at[b], gsem.at[b])

    def body(i, carry):
        j0 = i * DEPTH
        for b in range(DEPTH):
            j = j0 + b

            @pl.when(j < cnt)
            def _():
                pltpu.make_async_copy(
                    g_sh.at[sidx_v.at[j]], gbuf.at[b], gsem.at[b]
                ).wait()
                pltpu.sync_copy(gbuf.at[b], acc_sh.at[didx_v.at[j]], add=True)

            jn = j + DEPTH

            @pl.when(jn < cnt)
            def _():
                pltpu.async_copy(g_sh.at[sidx_v.at[jn]], gbuf.at[b], gsem.at[b])

        return carry

    lax.fori_loop(0, _ITER, body, 0)
    plsc.subcore_barrier()
    pltpu.sync_copy(degp_hbm.at[pl.ds(s * RPT, RPT)], d0_v)
    pltpu.sync_copy(degp_hbm.at[pl.ds(N_PAD + s * RPT, RPT)], d1_v)

    def dinv_loop(t, carry):
        d = d0_v[pl.ds(t * 16, 16)] + d1_v[pl.ds(t * 16, 16)] + 1.0
        yi = jnp.int32(0x5F3759DF) - (plsc.bitcast(d, jnp.int32) >> 1)
        y = plsc.bitcast(yi, jnp.float32)
        for _ in range(3):
            y = y * (1.5 - 0.5 * d * y * y)
        d0_v[pl.ds(t * 16, 16)] = y
        return carry

    lax.fori_loop(0, RPT // 16, dinv_loop, 0)

    for half in range(4):
        row0 = s * RPT + half * HRO
        pltpu.sync_copy(g_sh.at[pl.ds(row0, HRO), :], gv)
        pltpu.sync_copy(acc_sh.at[pl.ds(row0, HRO), :], av)

        def row_loop(r, carry, _half=half):
            idx = jnp.full((16,), _half * HRO, jnp.int32) + r
            dv = plsc.load_gather(d0_v, [idx])
            for k in range(HD // 16):
                t = gv[r, pl.ds(k * 16, 16)] + av[r, pl.ds(k * 16, 16)]
                gv[r, pl.ds(k * 16, 16)] = t * dv
            return carry

        lax.fori_loop(0, HRO, row_loop, 0)
        nvalid = TAIL_V[half]

        @pl.when(s < NS - 1)
        def _():
            pltpu.sync_copy(gv, out_hbm.at[pl.ds(row0, HRO), pl.ds(c * HD, HD)])

        if nvalid > 0:
            @pl.when(s == NS - 1)
            def _():
                pltpu.sync_copy(
                    gv.at[pl.ds(0, nvalid), :],
                    out_hbm.at[pl.ds(row0, nvalid), pl.ds(c * HD, HD)],
                )


_scat_kernel = pl.kernel(
    _scat_body,
    out_type=jax.ShapeDtypeStruct((N, D), jnp.float32),
    mesh=_mesh,
    scratch_types=[
        pltpu.VMEM((SC_Q + 1, CH), jnp.int32),
        pltpu.VMEM((SC_Q + 1, CH), jnp.int32),
        pltpu.VMEM((DEPTH, CH, HD), jnp.float32),
        pltpu.VMEM_SHARED((N_PAD, HD), jnp.float32),
        pltpu.VMEM_SHARED((N_PAD, HD), jnp.float32),
        pltpu.SemaphoreType.DMA((DEPTH,)),
        pltpu.VMEM((RPT,), jnp.float32),
        pltpu.VMEM((RPT,), jnp.float32),
        pltpu.VMEM((HRO, HD), jnp.float32),
        pltpu.VMEM((HRO, HD), jnp.float32),
    ],
    compiler_params=_sc_params_nl,
)


def _proj_body(x_ref, w_ref, b_ref, deg_ref, g0_ref, g1_ref):
    h = lax.dot_general(
        x_ref[...], w_ref[...], (((1,), (1,)), ((), ())),
        preferred_element_type=jnp.float32,
    )
    h = h + b_ref[...]
    nrm = jnp.sqrt(jnp.sum(h * h, axis=1, keepdims=True))
    h = (h / jnp.maximum(nrm, 1e-12)) * 1.8
    d = deg_ref[pl.ds(0, N), :] + deg_ref[pl.ds(N_PAD, N), :] + 1.0
    g = h * lax.rsqrt(d)
    g0_ref[pl.ds(0, N), :] = g[:, :HD]
    g1_ref[pl.ds(0, N), :] = g[:, HD:]
    tail = jnp.zeros((N_PAD - N, HD), jnp.float32)
    g0_ref[pl.ds(N, N_PAD - N), :] = tail
    g1_ref[pl.ds(N, N_PAD - N), :] = tail


_proj = pl.pallas_call(
    _proj_body,
    out_shape=(
        jax.ShapeDtypeStruct((N_PAD, HD), jnp.float32),
        jax.ShapeDtypeStruct((N_PAD, HD), jnp.float32),
    ),
)


def kernel(x, edge_index, W, b):
    e2d = edge_index.astype(jnp.int32).reshape(NC * NCHUNK, CH)
    ones = jnp.ones((CH,), jnp.float32)
    zeros1 = jnp.zeros((CH,), jnp.float32)
    zeros2 = jnp.zeros((CH, HD), jnp.float32)

    degp = _deg_kernel(e2d, ones, zeros1)
    degp2 = degp.reshape(NC * N_PAD, 1)
    g0, g1 = _proj(x, W, b.reshape(1, D), degp2)
    return _scat_kernel(e2d, g0, g1, zeros2, degp)


# speedup vs baseline: 1.2313x; 1.1011x over previous
import jax
import jax.numpy as jnp
from jax import lax
from jax.experimental import pallas as pl
from jax.experimental.pallas import tpu as pltpu
from jax.experimental.pallas import tpu_sc as plsc

N = 10000
IN_CH = 128
D = 64
HD = D // 2
E = 320000
NC = 2
NS = 16
NW = NC * NS
CH = 128
NCHUNK = E // CH
DEG_Q, DEG_R = divmod(NCHUNK, NW)
SC_Q, SC_R = divmod(NCHUNK, NS)
N_PAD = 10240
RPT = N_PAD // NS
HRO = RPT // 4
TAIL_V = tuple(max(0, min(HRO, N - ((NS - 1) * RPT + q * HRO))) for q in range(4))
DEPTH = 8
WIN = 16

_mesh = plsc.VectorSubcoreMesh(
    core_axis_name="c", subcore_axis_name="s", num_cores=NC, num_subcores=NS
)
_sc_params = pltpu.CompilerParams(use_tc_tiling_on_sc=False)
_sc_params_nl = pltpu.CompilerParams(
    use_tc_tiling_on_sc=False, needs_layout_passes=False
)


def _deg_body(e_hbm, ones_hbm, zeros_hbm, deg_out, idx_v, ones_v, deg_sh, dsem):
    c = lax.axis_index("c")
    s = lax.axis_index("s")
    wid = c * NS + s
    cnt = DEG_Q + jnp.where(wid < DEG_R, 1, 0)
    lo = NCHUNK + wid * DEG_Q + jnp.minimum(wid, DEG_R)
    pltpu.sync_copy(e_hbm.at[pl.ds(lo, DEG_Q), :], idx_v.at[pl.ds(0, DEG_Q), :])

    @pl.when(wid < DEG_R)
    def _():
        pltpu.sync_copy(e_hbm.at[pl.ds(lo + DEG_Q, 1), :],
                        idx_v.at[pl.ds(DEG_Q, 1), :])

    pltpu.sync_copy(ones_hbm, ones_v)
    for q in range(RPT // CH):
        pltpu.sync_copy(zeros_hbm, deg_sh.at[pl.ds(s * RPT + q * CH, CH)])
    plsc.subcore_barrier()

    def body(i, carry):
        for b in range(DEPTH):
            pltpu.async_copy(ones_v, deg_sh.at[idx_v.at[i * DEPTH + b]], dsem,
                             add=True)
        for b in range(DEPTH):
            pltpu.make_async_copy(
                ones_v, deg_sh.at[idx_v.at[i * DEPTH + b]], dsem
            ).wait()
        return carry

    lax.fori_loop(0, DEG_Q // DEPTH, body, 0)

    def tail(j, carry):
        pltpu.sync_copy(ones_v, deg_sh.at[idx_v.at[j]], add=True)
        return carry

    lax.fori_loop((DEG_Q // DEPTH) * DEPTH, cnt, tail, 0)
    plsc.subcore_barrier()
    pltpu.sync_copy(
        deg_sh.at[pl.ds(s * RPT, RPT)],
        deg_out.at[pl.ds(c * N_PAD + s * RPT, RPT)],
    )


_deg_kernel = pl.kernel(
    _deg_body,
    out_type=jax.ShapeDtypeStruct((NC * N_PAD,), jnp.float32),
    mesh=_mesh,
    scratch_types=[
        pltpu.VMEM((DEG_Q + 1, CH), jnp.int32),
        pltpu.VMEM((CH,), jnp.float32),
        pltpu.VMEM_SHARED((N_PAD,), jnp.float32),
        pltpu.SemaphoreType.DMA,
    ],
    compiler_params=_sc_params,
)

_ITER = -(-(SC_Q + 1) // DEPTH)


def _scat_body(e_hbm, g0_hbm, g1_hbm, zeros_hbm, degp_hbm, out_hbm,
               sidx_v, didx_v, gbuf, g_sh, acc_sh, gsem, d0_v, d1_v, gv, av):
    c = lax.axis_index("c")
    s = lax.axis_index("s")
    cnt = SC_Q + jnp.where(s < SC_R, 1, 0)
    lo = s * SC_Q + jnp.minimum(s, SC_R)
    pltpu.sync_copy(e_hbm.at[pl.ds(lo, SC_Q), :], sidx_v.at[pl.ds(0, SC_Q), :])
    pltpu.sync_copy(e_hbm.at[pl.ds(NCHUNK + lo, SC_Q), :],
                    didx_v.at[pl.ds(0, SC_Q), :])

    @pl.when(s < SC_R)
    def _():
        pltpu.sync_copy(e_hbm.at[pl.ds(lo + SC_Q, 1), :],
                        sidx_v.at[pl.ds(SC_Q, 1), :])
        pltpu.sync_copy(e_hbm.at[pl.ds(NCHUNK + lo + SC_Q, 1), :],
                        didx_v.at[pl.ds(SC_Q, 1), :])

    for q in range(RPT // CH):
        pltpu.sync_copy(zeros_hbm, acc_sh.at[pl.ds(s * RPT + q * CH, CH), :])
    pltpu.sync_copy(degp_hbm.at[pl.ds(s * RPT, RPT)], d0_v)
    pltpu.sync_copy(degp_hbm.at[pl.ds(N_PAD + s * RPT, RPT)], d1_v)

    def dinv_loop(t, carry):
        d = d0_v[pl.ds(t * 16, 16)] + d1_v[pl.ds(t * 16, 16)] + 1.0
        yi = jnp.int32(0x5F3759DF) - (plsc.bitcast(d, jnp.int32) >> 1)
        y = plsc.bitcast(yi, jnp.float32)
        for _ in range(3):
            y = y * (1.5 - 0.5 * d * y * y)
        d0_v[pl.ds(t * 16, 16)] = y
        return carry

    lax.fori_loop(0, RPT // 16, dinv_loop, 0)
    for part in range(4):
        row0 = s * RPT + part * HRO

        @pl.when(c == 0)
        def _():
            pltpu.sync_copy(g0_hbm.at[pl.ds(row0, HRO), :], gv)

        @pl.when(c == 1)
        def _():
            pltpu.sync_copy(g1_hbm.at[pl.ds(row0, HRO), :], gv)

        def scale_loop(r, carry, _part=part):
            idx = jnp.full((16,), _part * HRO, jnp.int32) + r
            dv = plsc.load_gather(d0_v, [idx])
            for k in range(HD // 16):
                gv[r, pl.ds(k * 16, 16)] = gv[r, pl.ds(k * 16, 16)] * dv
            return carry

        lax.fori_loop(0, HRO, scale_loop, 0)
        pltpu.sync_copy(gv, g_sh.at[pl.ds(row0, HRO), :])

    plsc.subcore_barrier()
    for b in range(DEPTH):
        pltpu.async_copy(g_sh.at[sidx_v.at[b]], gbuf.at[b], gsem.at[b])

    def body(i, carry):
        j0 = i * DEPTH
        for b in range(DEPTH):
            j = j0 + b

            @pl.when(j < cnt)
            def _():
                pltpu.make_async_copy(
                    g_sh.at[sidx_v.at[j]], gbuf.at[b], gsem.at[b]
                ).wait()
                pltpu.sync_copy(gbuf.at[b], acc_sh.at[didx_v.at[j]], add=True)

            jn = j + DEPTH

            @pl.when(jn < cnt)
            def _():
                pltpu.async_copy(g_sh.at[sidx_v.at[jn]], gbuf.at[b], gsem.at[b])

        return carry

    lax.fori_loop(0, _ITER, body, 0)
    plsc.subcore_barrier()
    for half in range(4):
        row0 = s * RPT + half * HRO
        pltpu.sync_copy(g_sh.at[pl.ds(row0, HRO), :], gv)
        pltpu.sync_copy(acc_sh.at[pl.ds(row0, HRO), :], av)

        def row_loop(r, carry, _half=half):
            idx = jnp.full((16,), _half * HRO, jnp.int32) + r
            dv = plsc.load_gather(d0_v, [idx])
            for k in range(HD // 16):
                t = gv[r, pl.ds(k * 16, 16)] + av[r, pl.ds(k * 16, 16)]
                gv[r, pl.ds(k * 16, 16)] = t * dv
            return carry

        lax.fori_loop(0, HRO, row_loop, 0)
        nvalid = TAIL_V[half]

        @pl.when(s < NS - 1)
        def _():
            pltpu.sync_copy(gv, out_hbm.at[pl.ds(row0, HRO), pl.ds(c * HD, HD)])

        if nvalid > 0:
            @pl.when(s == NS - 1)
            def _():
                pltpu.sync_copy(
                    gv.at[pl.ds(0, nvalid), :],
                    out_hbm.at[pl.ds(row0, nvalid), pl.ds(c * HD, HD)],
                )


_scat_kernel = pl.kernel(
    _scat_body,
    out_type=jax.ShapeDtypeStruct((N, D), jnp.float32),
    mesh=_mesh,
    scratch_types=[
        pltpu.VMEM((SC_Q + 1, CH), jnp.int32),
        pltpu.VMEM((SC_Q + 1, CH), jnp.int32),
        pltpu.VMEM((DEPTH, CH, HD), jnp.float32),
        pltpu.VMEM_SHARED((N_PAD, HD), jnp.float32),
        pltpu.VMEM_SHARED((N_PAD, HD), jnp.float32),
        pltpu.SemaphoreType.DMA((DEPTH,)),
        pltpu.VMEM((RPT,), jnp.float32),
        pltpu.VMEM((RPT,), jnp.float32),
        pltpu.VMEM((HRO, HD), jnp.float32),
        pltpu.VMEM((HRO, HD), jnp.float32),
    ],
    compiler_params=_sc_params_nl,
)


def _proj_body(x_ref, w_ref, b_ref, g0_ref, g1_ref):
    h = lax.dot_general(
        x_ref[...], w_ref[...], (((1,), (1,)), ((), ())),
        preferred_element_type=jnp.float32,
    )
    h = h + b_ref[...]
    nrm = jnp.sqrt(jnp.sum(h * h, axis=1, keepdims=True))
    g = (h / jnp.maximum(nrm, 1e-12)) * 1.8
    g0_ref[pl.ds(0, N), :] = g[:, :HD]
    g1_ref[pl.ds(0, N), :] = g[:, HD:]
    tail = jnp.zeros((N_PAD - N, HD), jnp.float32)
    g0_ref[pl.ds(N, N_PAD - N), :] = tail
    g1_ref[pl.ds(N, N_PAD - N), :] = tail


_proj = pl.pallas_call(
    _proj_body,
    out_shape=(
        jax.ShapeDtypeStruct((N_PAD, HD), jnp.float32),
        jax.ShapeDtypeStruct((N_PAD, HD), jnp.float32),
    ),
)


def kernel(x, edge_index, W, b):
    e2d = edge_index.astype(jnp.int32).reshape(NC * NCHUNK, CH)
    ones = jnp.ones((CH,), jnp.float32)
    zeros1 = jnp.zeros((CH,), jnp.float32)
    zeros2 = jnp.zeros((CH, HD), jnp.float32)

    degp = _deg_kernel(e2d, ones, zeros1)
    g0, g1 = _proj(x, W, b.reshape(1, D))
    return _scat_kernel(e2d, g0, g1, zeros2, degp)
